# Initial kernel scaffold; baseline (speedup 1.0000x reference)
import jax
import jax.numpy as jnp
from jax.experimental import pallas as pl


def kernel(indices, emb_weight):
    raise NotImplementedError("write your pallas kernel here")


# speedup vs baseline: 6.1040x; 6.1040x over previous
import functools
import math

import jax
import jax.numpy as jnp
import numpy as np
from jax import lax
from jax.experimental import pallas as pl
from jax.experimental.pallas import tpu as pltpu
from jax.experimental.pallas import tpu_sc as plsc

NUM_EMB = 100000
DIM = 32
BATCH = 16384
HIST = 200

NC = 2
NS = 16
NW = NC * NS

B_TOT = BATCH * HIST
B_PER_W = B_TOT // NW
CHUNK = 1024
N_CHUNK = B_PER_W // CHUNK
assert B_PER_W * NW == B_TOT and N_CHUNK * CHUNK == B_PER_W


def _pe_host() -> np.ndarray:
    position = np.arange(0, NUM_EMB, dtype=np.float32)[:, None]
    div_term = np.exp(
        np.arange(0, DIM, 2, dtype=np.float32) * (-(math.log(10000.0) / DIM))
    )
    pe = np.zeros((NUM_EMB, DIM), dtype=np.float32)
    pe[:, 0::2] = np.sin(position * div_term)
    pe[:, 1::2] = np.cos(position * div_term)
    return pe


_PE = _pe_host()


def _add_body(w_ref, pe_ref, out_ref):
    out_ref[...] = w_ref[...] + pe_ref[...]


def _table_add(emb_weight):
    w = emb_weight.reshape(25000, 128)
    pe = jnp.asarray(_PE.reshape(25000, 128))
    out = pl.pallas_call(
        _add_body,
        out_shape=jax.ShapeDtypeStruct((25000, 128), jnp.float32),
        grid=(5,),
        in_specs=[
            pl.BlockSpec((5000, 128), lambda i: (i, 0)),
            pl.BlockSpec((5000, 128), lambda i: (i, 0)),
        ],
        out_specs=pl.BlockSpec((5000, 128), lambda i: (i, 0)),
    )(w, pe)
    return out.reshape(NUM_EMB, DIM)


def _gather_body(table_hbm, idx_hbm, out_hbm, idx_v, rows_v, sem):
    wid = lax.axis_index("s") * NC + lax.axis_index("c")
    base = wid * B_PER_W

    def body(i, _):
        off = base + i * CHUNK
        pltpu.sync_copy(idx_hbm.at[pl.ds(off, CHUNK)], idx_v)
        pltpu.async_copy(table_hbm.at[idx_v], rows_v, sem).wait()
        pltpu.sync_copy(rows_v, out_hbm.at[pl.ds(off, CHUNK)])
        return 0

    lax.fori_loop(0, N_CHUNK, body, 0)


def _gather_sc(table, idx_flat):
    mesh = plsc.VectorSubcoreMesh(core_axis_name="c", subcore_axis_name="s")
    k = functools.partial(
        pl.kernel,
        mesh=mesh,
        out_type=jax.ShapeDtypeStruct((B_TOT, DIM), jnp.float32),
        scratch_types=[
            pltpu.VMEM((CHUNK,), jnp.int32),
            pltpu.VMEM((CHUNK, DIM), jnp.float32),
            pltpu.SemaphoreType.DMA,
        ],
        compiler_params=pltpu.CompilerParams(use_tc_tiling_on_sc=False),
    )(_gather_body)
    return k(table, idx_flat)


def kernel(indices, emb_weight):
    table = _table_add(emb_weight)
    idx_flat = indices.reshape(B_TOT).astype(jnp.int32)
    out = _gather_sc(table, idx_flat)
    return out.reshape(BATCH, HIST, DIM)


# speedup vs baseline: 6.4237x; 1.0524x over previous
import functools
import math

import jax
import jax.numpy as jnp
import numpy as np
from jax import lax
from jax.experimental import pallas as pl
from jax.experimental.pallas import tpu as pltpu
from jax.experimental.pallas import tpu_sc as plsc

NUM_EMB = 100000
DIM = 32
BATCH = 16384
HIST = 200

NC = 2
NS = 16
NW = NC * NS

B_TOT = BATCH * HIST
B_PER_W = B_TOT // NW
CHUNK = 1024
N_CHUNK = B_PER_W // CHUNK
assert B_PER_W * NW == B_TOT and N_CHUNK * CHUNK == B_PER_W


def _pe_host() -> np.ndarray:
    position = np.arange(0, NUM_EMB, dtype=np.float32)[:, None]
    div_term = np.exp(
        np.arange(0, DIM, 2, dtype=np.float32) * (-(math.log(10000.0) / DIM))
    )
    pe = np.zeros((NUM_EMB, DIM), dtype=np.float32)
    pe[:, 0::2] = np.sin(position * div_term)
    pe[:, 1::2] = np.cos(position * div_term)
    return pe


_PE = _pe_host()


def _add_body(w_ref, pe_ref, out_ref):
    out_ref[...] = w_ref[...] + pe_ref[...]


def _table_add(emb_weight):
    w = emb_weight.reshape(25000, 128)
    pe = jnp.asarray(_PE.reshape(25000, 128))
    out = pl.pallas_call(
        _add_body,
        out_shape=jax.ShapeDtypeStruct((25000, 128), jnp.float32),
        grid=(5,),
        in_specs=[
            pl.BlockSpec((5000, 128), lambda i: (i, 0)),
            pl.BlockSpec((5000, 128), lambda i: (i, 0)),
        ],
        out_specs=pl.BlockSpec((5000, 128), lambda i: (i, 0)),
    )(w, pe)
    return out.reshape(NUM_EMB, DIM)


NBUF = 2
assert N_CHUNK % NBUF == 0


def _gather_body(table_hbm, idx_hbm, out_hbm, idx_v, rows_v,
                 idx_s0, idx_s1, gat_s0, gat_s1, out_s0, out_s1):
    wid = lax.axis_index("s") * NC + lax.axis_index("c")
    base = wid * B_PER_W
    idx_sems = (idx_s0, idx_s1)
    gat_sems = (gat_s0, gat_s1)
    out_sems = (out_s0, out_s1)

    def idx_load(b, g):
        return pltpu.make_async_copy(
            idx_hbm.at[pl.ds(base + g * CHUNK, CHUNK)], idx_v.at[b], idx_sems[b])

    def gather(b):
        return pltpu.make_async_copy(
            table_hbm.at[idx_v.at[b]], rows_v.at[b], gat_sems[b])

    def writeback(b, g):
        return pltpu.make_async_copy(
            rows_v.at[b], out_hbm.at[pl.ds(base + g * CHUNK, CHUNK)], out_sems[b])

    for b in range(NBUF):
        idx_load(b, b).start()

    def outer(t, carry):
        go = t * NBUF
        for b in range(NBUF):
            g = go + b
            idx_load(b, g).wait()

            @pl.when(g >= NBUF)
            def _():
                writeback(b, g - NBUF).wait()

            gather(b).start()
            gather(b).wait()

            @pl.when(g + NBUF < N_CHUNK)
            def _():
                idx_load(b, g + NBUF).start()

            writeback(b, g).start()
        return carry

    lax.fori_loop(0, N_CHUNK // NBUF, outer, 0)

    for b in range(NBUF):
        writeback(b, 0).wait()


def _gather_sc(table, idx_flat):
    mesh = plsc.VectorSubcoreMesh(core_axis_name="c", subcore_axis_name="s")
    k = functools.partial(
        pl.kernel,
        mesh=mesh,
        out_type=jax.ShapeDtypeStruct((B_TOT, DIM), jnp.float32),
        scratch_types=[
            pltpu.VMEM((NBUF, CHUNK), jnp.int32),
            pltpu.VMEM((NBUF, CHUNK, DIM), jnp.float32),
            pltpu.SemaphoreType.DMA,
            pltpu.SemaphoreType.DMA,
            pltpu.SemaphoreType.DMA,
            pltpu.SemaphoreType.DMA,
            pltpu.SemaphoreType.DMA,
            pltpu.SemaphoreType.DMA,
        ],
        compiler_params=pltpu.CompilerParams(use_tc_tiling_on_sc=False),
    )(_gather_body)
    return k(table, idx_flat)


def kernel(indices, emb_weight):
    table = _table_add(emb_weight)
    idx_flat = indices.reshape(B_TOT).astype(jnp.int32)
    out = _gather_sc(table, idx_flat)
    return out.reshape(BATCH, HIST, DIM)


# speedup vs baseline: 6.4299x; 1.0010x over previous
import functools
import math

import jax
import jax.numpy as jnp
import numpy as np
from jax import lax
from jax.experimental import pallas as pl
from jax.experimental.pallas import tpu as pltpu
from jax.experimental.pallas import tpu_sc as plsc

NUM_EMB = 100000
DIM = 32
BATCH = 16384
HIST = 200

NC = 2
NS = 16
NW = NC * NS

B_TOT = BATCH * HIST
B_PER_W = B_TOT // NW
BATCH_PER_W = BATCH // NW
CHUNKB = 8
CHUNK = CHUNKB * HIST
N_CHUNK = BATCH_PER_W // CHUNKB
assert B_PER_W * NW == B_TOT and N_CHUNK * CHUNKB == BATCH_PER_W


def _pe_host() -> np.ndarray:
    position = np.arange(0, NUM_EMB, dtype=np.float32)[:, None]
    div_term = np.exp(
        np.arange(0, DIM, 2, dtype=np.float32) * (-(math.log(10000.0) / DIM))
    )
    pe = np.zeros((NUM_EMB, DIM), dtype=np.float32)
    pe[:, 0::2] = np.sin(position * div_term)
    pe[:, 1::2] = np.cos(position * div_term)
    return pe


_PE = _pe_host()


def _add_body(w_ref, pe_ref, out_ref):
    out_ref[...] = w_ref[...] + pe_ref[...]


def _table_add(emb_weight):
    w = emb_weight.reshape(25000, 128)
    pe = jnp.asarray(_PE.reshape(25000, 128))
    out = pl.pallas_call(
        _add_body,
        out_shape=jax.ShapeDtypeStruct((25000, 128), jnp.float32),
        grid=(5,),
        in_specs=[
            pl.BlockSpec((5000, 128), lambda i: (i, 0)),
            pl.BlockSpec((5000, 128), lambda i: (i, 0)),
        ],
        out_specs=pl.BlockSpec((5000, 128), lambda i: (i, 0)),
    )(w, pe)
    return out.reshape(NUM_EMB, DIM)


NBUF = 2
assert N_CHUNK % NBUF == 0


def _gather_body(table_hbm, idx_hbm, out_hbm, idx_v, rows_v,
                 idx_s0, idx_s1, gat_s0, gat_s1, out_s0, out_s1):
    wid = lax.axis_index("s") * NC + lax.axis_index("c")
    base = wid * B_PER_W
    bbase = wid * BATCH_PER_W
    idx_sems = (idx_s0, idx_s1)
    gat_sems = (gat_s0, gat_s1)
    out_sems = (out_s0, out_s1)

    def idx_load(b, g):
        return pltpu.make_async_copy(
            idx_hbm.at[pl.ds(base + g * CHUNK, CHUNK)], idx_v.at[b], idx_sems[b])

    def gather(b):
        return pltpu.make_async_copy(
            table_hbm.at[idx_v.at[b]], rows_v.at[b], gat_sems[b])

    def writeback(b, g):
        def one(j):
            return pltpu.make_async_copy(
                rows_v.at[b, pl.ds(j * HIST, HIST)],
                out_hbm.at[bbase + g * CHUNKB + j],
                out_sems[b])

        return one

    for b in range(NBUF):
        idx_load(b, b).start()

    def outer(t, carry):
        go = t * NBUF
        for b in range(NBUF):
            g = go + b
            idx_load(b, g).wait()

            @pl.when(g >= NBUF)
            def _():
                for j in range(CHUNKB):
                    writeback(b, 0)(j).wait()

            gather(b).start()
            gather(b).wait()

            @pl.when(g + NBUF < N_CHUNK)
            def _():
                idx_load(b, g + NBUF).start()

            wb = writeback(b, g)
            for j in range(CHUNKB):
                wb(j).start()
        return carry

    lax.fori_loop(0, N_CHUNK // NBUF, outer, 0)

    for b in range(NBUF):
        for j in range(CHUNKB):
            writeback(b, 0)(j).wait()


def _gather_sc(table, idx_flat):
    mesh = plsc.VectorSubcoreMesh(core_axis_name="c", subcore_axis_name="s")
    k = functools.partial(
        pl.kernel,
        mesh=mesh,
        out_type=jax.ShapeDtypeStruct((BATCH, HIST, DIM), jnp.float32),
        scratch_types=[
            pltpu.VMEM((NBUF, CHUNK), jnp.int32),
            pltpu.VMEM((NBUF, CHUNK, DIM), jnp.float32),
            pltpu.SemaphoreType.DMA,
            pltpu.SemaphoreType.DMA,
            pltpu.SemaphoreType.DMA,
            pltpu.SemaphoreType.DMA,
            pltpu.SemaphoreType.DMA,
            pltpu.SemaphoreType.DMA,
        ],
        compiler_params=pltpu.CompilerParams(use_tc_tiling_on_sc=False),
    )(_gather_body)
    return k(table, idx_flat)


def kernel(indices, emb_weight):
    table = _table_add(emb_weight)
    idx_flat = indices.reshape(B_TOT).astype(jnp.int32)
    return _gather_sc(table, idx_flat)
